# baseline (speedup 1.0000x reference)
import functools

import jax
import jax.numpy as jnp
from jax.experimental import pallas as pl
from jax.experimental.pallas import tpu as pltpu
from jax.experimental.pallas import tpu_sc as plsc

_MAX_POS = 8192
_EMBEDDING_DIM = 16
_TILE_ROWS = 8


@functools.partial(
    pl.kernel,
    out_type=jax.ShapeDtypeStruct((_EMBEDDING_DIM, _MAX_POS), jnp.float32),
    mesh=plsc.ScalarSubcoreMesh(axis_name="c", num_cores=1),
    scratch_types=[
        pltpu.MemorySpace.VMEM_SHARED((_TILE_ROWS, _MAX_POS), jnp.float32),
        pltpu.MemorySpace.VMEM_SHARED((_TILE_ROWS, _MAX_POS), jnp.float32),
        pltpu.SemaphoreType.DMA,
        pltpu.SemaphoreType.DMA,
    ],
    compiler_params=pltpu.CompilerParams(
        use_tc_tiling_on_sc=True, skip_device_barrier=True
    ),
)
def _sc_row_copy(wt_hbm, out_hbm, buf0, buf1, sem0, sem1):
    lo = pl.ds(0, _TILE_ROWS)
    hi = pl.ds(_TILE_ROWS, _TILE_ROWS)
    in0 = pltpu.async_copy(wt_hbm.at[lo], buf0, sem0)
    in1 = pltpu.async_copy(wt_hbm.at[hi], buf1, sem1)
    in0.wait()
    out0 = pltpu.async_copy(buf0, out_hbm.at[lo], sem0)
    in1.wait()
    out1 = pltpu.async_copy(buf1, out_hbm.at[hi], sem1)
    out0.wait()
    out1.wait()


def kernel(input, weights):
    del input
    return _sc_row_copy(weights.T).T
